# Initial kernel scaffold; baseline (speedup 1.0000x reference)
import jax
import jax.numpy as jnp
from jax.experimental import pallas as pl


def kernel(xs, W_in, b_in, blk_ln_g, blk_ln_b, blk_Wuv, blk_buv, blk_Wz, blk_bz, blk_gam, blk_bet, blk_Wo, blk_bo, tr_g, tr_b, W_tr, b_tr, out_g, out_b, W_out, b_out):
    raise NotImplementedError("write your pallas kernel here")


# speedup vs baseline: 1.0309x; 1.0309x over previous
import functools

import numpy as np
import jax
import jax.numpy as jnp
from jax.experimental import pallas as pl

D = 128
E = 256
S = 64
BUCKET = 64
N_BLOCK = 2


def _dot(a, b, dims):
    return jax.lax.dot_general(a, b, (dims, ((), ())),
                               preferred_element_type=jnp.float32)


def _proj_body(xs_ref, w_ref, b_ref, o_ref):
    o_ref[0] = _dot(xs_ref[0], w_ref[...], ((1,), (0,))) + b_ref[...]


def _proj(xs, W_in, b_in):
    Bn, n0, din = xs.shape
    RB = 2048
    return pl.pallas_call(
        _proj_body,
        grid=(Bn, n0 // RB),
        in_specs=[
            pl.BlockSpec((1, RB, din), lambda s, r: (s, r, 0)),
            pl.BlockSpec((din, D), lambda s, r: (0, 0)),
            pl.BlockSpec((1, D), lambda s, r: (0, 0)),
        ],
        out_specs=pl.BlockSpec((1, RB, D), lambda s, r: (s, r, 0)),
        out_shape=jax.ShapeDtypeStruct((Bn, n0, D), jnp.float32),
    )(xs, W_in, b_in.reshape(1, D))


def _cw_body(x_ref, cw_ref):
    x = x_ref[0]
    n = x.shape[0]
    ones = jnp.ones((1, D), jnp.float32)
    lens2 = _dot(ones, x * x, ((1,), (1,)))
    m = jnp.max(lens2)
    iota = jax.lax.broadcasted_iota(jnp.int32, (1, n), 1)
    idx = jnp.min(jnp.where(lens2 == m, iota, n))
    onehot = (iota == idx).astype(jnp.float32)
    v1 = _dot(onehot, x, ((1,), (0,)))
    v1n = jnp.sqrt(jnp.sum(v1 * v1))
    dots = _dot(v1, x, ((1,), (1,)))
    denom = jnp.maximum(v1n * jnp.sqrt(lens2), 1e-8)
    cw_ref[0] = dots / denom


def _cosine_weights(x):
    Bn, n, _ = x.shape
    out = pl.pallas_call(
        _cw_body,
        grid=(Bn,),
        in_specs=[pl.BlockSpec((1, n, D), lambda s: (s, 0, 0))],
        out_specs=pl.BlockSpec((1, 1, n), lambda s: (s, 0, 0)),
        out_shape=jax.ShapeDtypeStruct((Bn, 1, n), jnp.float32),
    )(x)
    return out[:, 0]


def _gau_body(G, g_ref, lng_ref, lnb_ref, wuv_ref, buv_ref, wz_ref, bz_ref,
              gam_ref, bet_ref, wo_ref, bo_ref, trg_ref, trb_ref, wtr_ref,
              btr_ref, gys_ref, ys_ref):
    x = g_ref[0]
    for j in range(N_BLOCK):
        mean = jnp.mean(x, axis=1, keepdims=True)
        var = jnp.mean((x - mean) ** 2, axis=1, keepdims=True)
        xn = (x - mean) / jnp.sqrt(var + 1e-5) * lng_ref[j:j + 1] \
            + lnb_ref[j:j + 1]
        uv = _dot(xn, wuv_ref[j], ((1,), (0,))) + buv_ref[j:j + 1]
        uv = uv * jax.nn.sigmoid(uv)
        u = uv[:, :E]
        v = uv[:, E:]
        z = _dot(xn, wz_ref[j], ((1,), (0,))) + bz_ref[j:j + 1]
        q = z * gam_ref[2 * j:2 * j + 1] + bet_ref[2 * j:2 * j + 1]
        k = z * gam_ref[2 * j + 1:2 * j + 2] + bet_ref[2 * j + 1:2 * j + 2]
        outs = []
        for t in range(G):
            sl = slice(t * BUCKET, (t + 1) * BUCKET)
            qk = _dot(q[sl], k[sl], ((1,), (1,))) * (1.0 / BUCKET)
            A = jnp.square(jnp.maximum(qk, 0.0))
            outs.append(_dot(A, v[sl], ((1,), (0,))))
        av = jnp.concatenate(outs, axis=0) if G > 1 else outs[0]
        x = x + _dot(u * av, wo_ref[j], ((1,), (0,))) + bo_ref[j:j + 1]
    rows = [jnp.mean(x[t * BUCKET:(t + 1) * BUCKET], axis=0, keepdims=True)
            for t in range(G)]
    rows = jnp.concatenate(rows, axis=0) if G > 1 else rows[0]
    gys_ref[0] = rows
    m2 = jnp.mean(rows, axis=1, keepdims=True)
    v2 = jnp.mean((rows - m2) ** 2, axis=1, keepdims=True)
    t_ = (rows - m2) / jnp.sqrt(v2 + 1e-5) * trg_ref[...] + trb_ref[...]
    t_ = jnp.where(t_ >= 0, t_, 0.01 * t_)
    ys_ref[0] = _dot(t_, wtr_ref[...], ((1,), (0,))) + btr_ref[...]


def _gau_level(g, G, wts):
    (lng, lnb, wuv, buv, wz, bz, gam, bet, wo, bo, trg, trb, wtr, btr) = wts
    Bn, rows, _ = g.shape
    nbp = rows // BUCKET
    grid = (Bn, nbp // G)
    const3 = lambda shp: pl.BlockSpec(shp, lambda s, b: (0, 0, 0))
    const2 = lambda shp: pl.BlockSpec(shp, lambda s, b: (0, 0))
    return pl.pallas_call(
        functools.partial(_gau_body, G),
        grid=grid,
        in_specs=[
            pl.BlockSpec((1, G * BUCKET, D), lambda s, b: (s, b, 0)),
            const2((N_BLOCK, D)), const2((N_BLOCK, D)),
            const3((N_BLOCK, D, 2 * E)), const2((N_BLOCK, 2 * E)),
            const3((N_BLOCK, D, S)), const2((N_BLOCK, S)),
            const2((2 * N_BLOCK, S)), const2((2 * N_BLOCK, S)),
            const3((N_BLOCK, E, D)), const2((N_BLOCK, D)),
            const2((1, D)), const2((1, D)),
            const2((D, D)), const2((1, D)),
        ],
        out_specs=[
            pl.BlockSpec((1, G, D), lambda s, b: (s, b, 0)),
            pl.BlockSpec((1, G, D), lambda s, b: (s, b, 0)),
        ],
        out_shape=[
            jax.ShapeDtypeStruct((Bn, nbp, D), jnp.float32),
            jax.ShapeDtypeStruct((Bn, nbp, D), jnp.float32),
        ],
    )(g, lng, lnb, wuv, buv, wz, bz, gam, bet, wo, bo, trg, trb, wtr, btr)


def _final_body(ys_ref, og_ref, ob_ref, wout_ref, bout_ref, o_ref):
    Bn = ys_ref.shape[0]
    rows = [jnp.mean(ys_ref[i], axis=0, keepdims=True) for i in range(Bn)]
    y = jnp.concatenate(rows, axis=0)
    m = jnp.mean(y, axis=1, keepdims=True)
    v = jnp.mean((y - m) ** 2, axis=1, keepdims=True)
    y = (y - m) / jnp.sqrt(v + 1e-5) * og_ref[...] + ob_ref[...]
    y = jnp.where(y >= 0, y, 0.01 * y)
    o_ref[...] = _dot(y, wout_ref[...], ((1,), (0,))) + bout_ref[...]


def _final(ys, out_g, out_b, W_out, b_out):
    Bn, nrows, _ = ys.shape
    od = W_out.shape[1]
    return pl.pallas_call(
        _final_body,
        in_specs=[
            pl.BlockSpec((Bn, nrows, D), lambda: (0, 0, 0)),
            pl.BlockSpec((1, D), lambda: (0, 0)),
            pl.BlockSpec((1, D), lambda: (0, 0)),
            pl.BlockSpec((D, od), lambda: (0, 0)),
            pl.BlockSpec((1, od), lambda: (0, 0)),
        ],
        out_specs=pl.BlockSpec((Bn, od), lambda: (0, 0)),
        out_shape=jax.ShapeDtypeStruct((Bn, od), jnp.float32),
    )(ys, out_g.reshape(1, D), out_b.reshape(1, D), W_out,
      b_out.reshape(1, od))


def _pad_rows(n_pad):
    ids = np.arange(n_pad)
    pad = np.zeros((n_pad, D), np.float32)
    pad[ids, ids % D] = 1.0
    return jnp.asarray(pad)


def kernel(xs, W_in, b_in, blk_ln_g, blk_ln_b, blk_Wuv, blk_buv, blk_Wz,
           blk_bz, blk_gam, blk_bet, blk_Wo, blk_bo, tr_g, tr_b, W_tr, b_tr,
           out_g, out_b, W_out, b_out):
    Bn = xs.shape[0]
    wts = (blk_ln_g, blk_ln_b, blk_Wuv, blk_buv, blk_Wz, blk_bz,
           blk_gam.reshape(2 * N_BLOCK, S), blk_bet.reshape(2 * N_BLOCK, S),
           blk_Wo, blk_bo, tr_g.reshape(1, D), tr_b.reshape(1, D), W_tr,
           b_tr.reshape(1, D))

    x = _proj(xs, W_in, b_in)
    ys_list = []
    while True:
        n = x.shape[1]
        cw = _cosine_weights(x)
        order = jnp.argsort(-cw, axis=1)
        xs_sorted = jnp.take_along_axis(x, order[..., None], axis=1)
        n_pad = BUCKET - n % BUCKET
        pad = jnp.broadcast_to(_pad_rows(n_pad)[None], (Bn, n_pad, D))
        g = jnp.concatenate([xs_sorted, pad], axis=1)
        n_bucket = (n + n_pad) // BUCKET
        if n_bucket > 8:
            G = 8
            nbp = -(-n_bucket // G) * G
            extra = jnp.zeros((Bn, (nbp - n_bucket) * BUCKET, D), jnp.float32)
            g = jnp.concatenate([g, extra], axis=1)
        else:
            G = n_bucket
        gys, ys_l = _gau_level(g, G, wts)
        ys_list.append(ys_l[:, :n_bucket])
        if n_bucket == 1:
            break
        x = gys[:, :n_bucket]

    ys = jnp.concatenate(ys_list, axis=1)
    return _final(ys, out_g, out_b, W_out, b_out)
